# baseline (speedup 1.0000x reference)
import functools

import jax
import jax.numpy as jnp
from jax import lax
from jax.experimental import pallas as pl
from jax.experimental.pallas import tpu as pltpu
from jax.experimental.pallas import tpu_sc as plsc

_K = 5
_LANES = 16


def _make_sc_gather(batch, num_classes):
    mesh = plsc.VectorSubcoreMesh(core_axis_name="c", subcore_axis_name="s")
    num_workers = batch // _LANES

    @functools.partial(
        pl.kernel,
        out_type=jax.ShapeDtypeStruct((batch,), jnp.float32),
        mesh=mesh,
        scratch_types=[
            pltpu.VMEM((_LANES,), jnp.int32),
            pltpu.VMEM((_LANES, 8, 128), jnp.float32),
            pltpu.VMEM((_LANES,), jnp.float32),
            pltpu.SemaphoreType.DMA,
        ],
        compiler_params=pltpu.CompilerParams(
            use_tc_tiling_on_sc=True,
            needs_layout_passes=False,
            skip_device_barrier=True,
            disable_semaphore_checks=True,
        ),
    )
    def gather_kernel(prob_hbm, label_hbm, out_hbm, lab_v, win_v, val_v, sem):
        cid = lax.axis_index("c")
        sid = lax.axis_index("s")
        wid = sid * 2 + cid

        @pl.when(wid < num_workers)
        def _():
            base = wid * _LANES
            pltpu.sync_copy(label_hbm.at[pl.ds(base, _LANES)], lab_v)
            lab = lab_v[...]
            lanes = lax.iota(jnp.int32, _LANES)
            copies = []
            for r in range(_LANES):
                lab_r = jnp.max(jnp.where(lanes == r, lab, 0))
                j0 = pl.multiple_of(jnp.bitwise_and(lab_r, ~127), 128)
                band = pl.multiple_of(base + (r & ~7), 8)
                copies.append(
                    pltpu.async_copy(
                        prob_hbm.at[pl.ds(band, 8), pl.ds(j0, 128)],
                        win_v.at[r],
                        sem,
                    )
                )
            for c in copies:
                c.wait()
            vals = plsc.load_gather(
                win_v,
                [lanes, jnp.bitwise_and(lanes, 7), jnp.bitwise_and(lab, 127)],
            )
            val_v[...] = vals
            pltpu.sync_copy(val_v, out_hbm.at[pl.ds(base, _LANES)])

    return gather_kernel


def _make_tc_count(batch, num_classes, blk):
    grid = (num_classes // blk,)

    def body(prob_ref, v_ref, out_ref, acc_ref):
        i = pl.program_id(0)

        @pl.when(i == 0)
        def _():
            acc_ref[...] = jnp.zeros_like(acc_ref)

        ge = (prob_ref[...] >= v_ref[...]).astype(jnp.float32)
        acc_ref[...] = acc_ref[...] + jnp.sum(ge, axis=1, keepdims=True)

        @pl.when(i == grid[0] - 1)
        def _():
            hits = (acc_ref[...] <= float(_K)).astype(jnp.float32)
            out_ref[0, 0] = jnp.sum(hits) / float(batch)

    return pl.pallas_call(
        body,
        grid=grid,
        in_specs=[
            pl.BlockSpec((batch, blk), lambda i: (0, i)),
            pl.BlockSpec((batch, 1), lambda i: (0, 0)),
        ],
        out_specs=pl.BlockSpec((1, 1), lambda i: (0, 0), memory_space=pltpu.SMEM),
        out_shape=jax.ShapeDtypeStruct((1, 1), jnp.float32),
        scratch_shapes=[pltpu.VMEM((batch, 1), jnp.float32)],
    )


@jax.jit
def kernel(prob, label):
    batch, num_classes = prob.shape
    v = _make_sc_gather(batch, num_classes)(prob, label)
    out = _make_tc_count(batch, num_classes, 2048)(prob, v.reshape(batch, 1))
    return out[0, 0]


# speedup vs baseline: 1.0517x; 1.0517x over previous
import functools

import jax
import jax.numpy as jnp
from jax import lax
from jax.experimental import pallas as pl
from jax.experimental.pallas import tpu as pltpu
from jax.experimental.pallas import tpu_sc as plsc

_K = 5
_LANES = 16


def _make_sc_gather(batch, num_classes):
    num_workers = batch // _LANES
    mesh = plsc.VectorSubcoreMesh(
        core_axis_name="c", subcore_axis_name="s", num_cores=1,
        num_subcores=num_workers,
    )

    @functools.partial(
        pl.kernel,
        out_type=jax.ShapeDtypeStruct((batch,), jnp.float32),
        mesh=mesh,
        scratch_types=[
            pltpu.VMEM((_LANES,), jnp.int32),
            pltpu.VMEM((_LANES, 8, 128), jnp.float32),
            pltpu.VMEM((_LANES,), jnp.float32),
            pltpu.SemaphoreType.DMA,
        ],
        compiler_params=pltpu.CompilerParams(
            use_tc_tiling_on_sc=True,
            needs_layout_passes=False,
            skip_device_barrier=True,
            disable_semaphore_checks=True,
        ),
    )
    def gather_kernel(prob_hbm, label_hbm, out_hbm, lab_v, win_v, val_v, sem):
        cid = lax.axis_index("c")
        sid = lax.axis_index("s")
        wid = sid + cid * num_workers

        @pl.when(wid < num_workers)
        def _():
            base = wid * _LANES
            pltpu.sync_copy(label_hbm.at[pl.ds(base, _LANES)], lab_v)
            lab = lab_v[...]
            lanes = lax.iota(jnp.int32, _LANES)
            copies = []
            for r in range(_LANES):
                lab_r = jnp.max(jnp.where(lanes == r, lab, 0))
                j0 = pl.multiple_of(jnp.bitwise_and(lab_r, ~127), 128)
                band = pl.multiple_of(base + (r & ~7), 8)
                copies.append(
                    pltpu.async_copy(
                        prob_hbm.at[pl.ds(band, 8), pl.ds(j0, 128)],
                        win_v.at[r],
                        sem,
                    )
                )
            for c in copies:
                c.wait()
            vals = plsc.load_gather(
                win_v,
                [lanes, jnp.bitwise_and(lanes, 7), jnp.bitwise_and(lab, 127)],
            )
            val_v[...] = vals
            pltpu.sync_copy(val_v, out_hbm.at[pl.ds(base, _LANES)])

    return gather_kernel


def _make_tc_count(batch, num_classes, blk):
    grid = (num_classes // blk,)

    def body(prob_ref, v_ref, out_ref, acc_ref):
        i = pl.program_id(0)

        @pl.when(i == 0)
        def _():
            acc_ref[...] = jnp.zeros_like(acc_ref)

        ge = (prob_ref[...] >= v_ref[...]).astype(jnp.float32)
        acc_ref[...] = acc_ref[...] + jnp.sum(ge, axis=1, keepdims=True)

        @pl.when(i == grid[0] - 1)
        def _():
            hits = (acc_ref[...] <= float(_K)).astype(jnp.float32)
            out_ref[0, 0] = jnp.sum(hits) / float(batch)

    return pl.pallas_call(
        body,
        grid=grid,
        in_specs=[
            pl.BlockSpec((batch, blk), lambda i: (0, i)),
            pl.BlockSpec((batch, 1), lambda i: (0, 0)),
        ],
        out_specs=pl.BlockSpec((1, 1), lambda i: (0, 0), memory_space=pltpu.SMEM),
        out_shape=jax.ShapeDtypeStruct((1, 1), jnp.float32),
        scratch_shapes=[pltpu.VMEM((batch, 1), jnp.float32)],
    )


@jax.jit
def kernel(prob, label):
    batch, num_classes = prob.shape
    v = _make_sc_gather(batch, num_classes)(prob, label)
    out = _make_tc_count(batch, num_classes, 2048)(prob, v.reshape(batch, 1))
    return out[0, 0]


# speedup vs baseline: 1.1942x; 1.1355x over previous
import functools

import jax
import jax.numpy as jnp
from jax import lax
from jax.experimental import pallas as pl
from jax.experimental.pallas import tpu as pltpu
from jax.experimental.pallas import tpu_sc as plsc

_K = 5
_LANES = 16


def _make_sc_gather(batch, num_classes):
    num_workers = batch // _LANES
    mesh = plsc.VectorSubcoreMesh(
        core_axis_name="c", subcore_axis_name="s", num_cores=1,
        num_subcores=num_workers,
    )

    @functools.partial(
        pl.kernel,
        out_type=jax.ShapeDtypeStruct((batch,), jnp.float32),
        mesh=mesh,
        scratch_types=[
            pltpu.VMEM((_LANES,), jnp.int32),
            pltpu.VMEM((_LANES, 8, 128), jnp.float32),
            pltpu.VMEM((_LANES,), jnp.float32),
            pltpu.SemaphoreType.DMA,
        ],
        compiler_params=pltpu.CompilerParams(
            use_tc_tiling_on_sc=True,
            needs_layout_passes=False,
            skip_device_barrier=True,
            disable_semaphore_checks=True,
        ),
    )
    def gather_kernel(prob_hbm, label_hbm, out_hbm, lab_v, win_v, val_v, sem):
        cid = lax.axis_index("c")
        sid = lax.axis_index("s")
        wid = sid + cid * num_workers

        @pl.when(wid < num_workers)
        def _():
            base = wid * _LANES
            pltpu.sync_copy(label_hbm.at[pl.ds(base, _LANES)], lab_v)
            lab = lab_v[...]
            lanes = lax.iota(jnp.int32, _LANES)
            copies = []
            for r in range(_LANES):
                lab_r = jnp.max(jnp.where(lanes == r, lab, 0))
                j0 = pl.multiple_of(jnp.bitwise_and(lab_r, ~127), 128)
                band = pl.multiple_of(base + (r & ~7), 8)
                copies.append(
                    pltpu.async_copy(
                        prob_hbm.at[pl.ds(band, 8), pl.ds(j0, 128)],
                        win_v.at[r],
                        sem,
                    )
                )
            for c in copies:
                c.wait()
            vals = plsc.load_gather(
                win_v,
                [lanes, jnp.bitwise_and(lanes, 7), jnp.bitwise_and(lab, 127)],
            )
            val_v[...] = vals
            pltpu.sync_copy(val_v, out_hbm.at[pl.ds(base, _LANES)])

    return gather_kernel


def _make_tc_count(batch, num_classes, blk):
    grid = (num_classes // blk,)

    def body(prob_ref, v_ref, out_ref, acc_ref):
        i = pl.program_id(0)

        @pl.when(i == 0)
        def _():
            acc_ref[...] = jnp.zeros_like(acc_ref)

        ge = (prob_ref[...] >= v_ref[...]).astype(jnp.float32)
        acc_ref[...] = acc_ref[...] + jnp.sum(ge, axis=1, keepdims=True)

        @pl.when(i == grid[0] - 1)
        def _():
            hits = (acc_ref[...] <= float(_K)).astype(jnp.float32)
            out_ref[0, 0] = jnp.sum(hits) / float(batch)

    return pl.pallas_call(
        body,
        grid=grid,
        in_specs=[
            pl.BlockSpec((batch, blk), lambda i: (0, i)),
            pl.BlockSpec((batch, 1), lambda i: (0, 0)),
        ],
        out_specs=pl.BlockSpec((1, 1), lambda i: (0, 0), memory_space=pltpu.SMEM),
        out_shape=jax.ShapeDtypeStruct((1, 1), jnp.float32),
        scratch_shapes=[pltpu.VMEM((batch, 1), jnp.float32)],
    )


@jax.jit
def kernel(prob, label):
    batch, num_classes = prob.shape
    v = _make_sc_gather(batch, num_classes)(prob, label)
    out = _make_tc_count(batch, num_classes, 4096)(prob, v.reshape(batch, 1))
    return out[0, 0]


# speedup vs baseline: 1.2723x; 1.0654x over previous
import functools

import jax
import jax.numpy as jnp
from jax import lax
from jax.experimental import pallas as pl
from jax.experimental.pallas import tpu as pltpu
from jax.experimental.pallas import tpu_sc as plsc

_K = 5
_LANES = 16


def _make_sc_gather(batch, num_classes):
    num_workers = batch // _LANES
    mesh = plsc.VectorSubcoreMesh(
        core_axis_name="c", subcore_axis_name="s", num_cores=1,
        num_subcores=num_workers,
    )

    @functools.partial(
        pl.kernel,
        out_type=jax.ShapeDtypeStruct((batch,), jnp.float32),
        mesh=mesh,
        scratch_types=[
            pltpu.VMEM((_LANES,), jnp.int32),
            pltpu.VMEM((_LANES, 8, 128), jnp.float32),
            pltpu.VMEM((_LANES,), jnp.float32),
            pltpu.SemaphoreType.DMA,
        ],
        compiler_params=pltpu.CompilerParams(
            use_tc_tiling_on_sc=True,
            needs_layout_passes=False,
            skip_device_barrier=True,
            disable_semaphore_checks=True,
        ),
    )
    def gather_kernel(prob_hbm, label_hbm, out_hbm, lab_v, win_v, val_v, sem):
        cid = lax.axis_index("c")
        sid = lax.axis_index("s")
        wid = sid + cid * num_workers

        @pl.when(wid < num_workers)
        def _():
            base = wid * _LANES
            pltpu.sync_copy(label_hbm.at[pl.ds(base, _LANES)], lab_v)
            lab = lab_v[...]
            lanes = lax.iota(jnp.int32, _LANES)
            copies = []
            for r in range(_LANES):
                lab_r = jnp.max(jnp.where(lanes == r, lab, 0))
                j0 = pl.multiple_of(jnp.bitwise_and(lab_r, ~127), 128)
                band = pl.multiple_of(base + (r & ~7), 8)
                copies.append(
                    pltpu.async_copy(
                        prob_hbm.at[pl.ds(band, 8), pl.ds(j0, 128)],
                        win_v.at[r],
                        sem,
                    )
                )
            for c in copies:
                c.wait()
            vals = plsc.load_gather(
                win_v,
                [lanes, jnp.bitwise_and(lanes, 7), jnp.bitwise_and(lab, 127)],
            )
            val_v[...] = vals
            pltpu.sync_copy(val_v, out_hbm.at[pl.ds(base, _LANES)])

    return gather_kernel


def _make_tc_count(batch, num_classes, blk):
    grid = (num_classes // blk,)

    def body(prob_ref, v_ref, out_ref, acc_ref):
        i = pl.program_id(0)

        @pl.when(i == 0)
        def _():
            acc_ref[...] = jnp.zeros_like(acc_ref)

        ge = (prob_ref[...] >= v_ref[...]).astype(jnp.float32)
        acc_ref[...] = acc_ref[...] + jnp.sum(ge, axis=1, keepdims=True)

        @pl.when(i == grid[0] - 1)
        def _():
            hits = (acc_ref[...] <= float(_K)).astype(jnp.float32)
            out_ref[0, 0] = jnp.sum(hits) / float(batch)

    return pl.pallas_call(
        body,
        grid=grid,
        in_specs=[
            pl.BlockSpec((batch, blk), lambda i: (0, i)),
            pl.BlockSpec((batch, 1), lambda i: (0, 0)),
        ],
        out_specs=pl.BlockSpec((1, 1), lambda i: (0, 0), memory_space=pltpu.SMEM),
        out_shape=jax.ShapeDtypeStruct((1, 1), jnp.float32),
        scratch_shapes=[pltpu.VMEM((batch, 1), jnp.float32)],
    )


@jax.jit
def kernel(prob, label):
    batch, num_classes = prob.shape
    v = _make_sc_gather(batch, num_classes)(prob, label)
    out = _make_tc_count(batch, num_classes, 8192)(prob, v.reshape(batch, 1))
    return out[0, 0]


# speedup vs baseline: 1.2771x; 1.0038x over previous
import functools

import jax
import jax.numpy as jnp
from jax import lax
from jax.experimental import pallas as pl
from jax.experimental.pallas import tpu as pltpu
from jax.experimental.pallas import tpu_sc as plsc

_K = 5
_LANES = 16


def _make_sc_gather(batch, num_classes):
    num_workers = batch // _LANES
    mesh = plsc.VectorSubcoreMesh(
        core_axis_name="c", subcore_axis_name="s", num_cores=1,
        num_subcores=num_workers,
    )

    @functools.partial(
        pl.kernel,
        out_type=jax.ShapeDtypeStruct((batch,), jnp.float32),
        mesh=mesh,
        scratch_types=[
            pltpu.VMEM((_LANES,), jnp.int32),
            pltpu.VMEM((_LANES, 8, 128), jnp.float32),
            pltpu.VMEM((_LANES,), jnp.float32),
            pltpu.SemaphoreType.DMA,
        ],
        compiler_params=pltpu.CompilerParams(
            use_tc_tiling_on_sc=True,
            needs_layout_passes=False,
            skip_device_barrier=True,
            disable_semaphore_checks=True,
        ),
    )
    def gather_kernel(prob_hbm, label_hbm, out_hbm, lab_v, win_v, val_v, sem):
        cid = lax.axis_index("c")
        sid = lax.axis_index("s")
        wid = sid + cid * num_workers

        @pl.when(wid < num_workers)
        def _():
            base = wid * _LANES
            pltpu.sync_copy(label_hbm.at[pl.ds(base, _LANES)], lab_v)
            lab = lab_v[...]
            lanes = lax.iota(jnp.int32, _LANES)
            copies = []
            for r in range(_LANES):
                lab_r = jnp.max(jnp.where(lanes == r, lab, 0))
                j0 = pl.multiple_of(jnp.bitwise_and(lab_r, ~127), 128)
                band = pl.multiple_of(base + (r & ~7), 8)
                copies.append(
                    pltpu.async_copy(
                        prob_hbm.at[pl.ds(band, 8), pl.ds(j0, 128)],
                        win_v.at[r],
                        sem,
                    )
                )
            for c in copies:
                c.wait()
            vals = plsc.load_gather(
                win_v,
                [lanes, jnp.bitwise_and(lanes, 7), jnp.bitwise_and(lab, 127)],
            )
            val_v[...] = vals
            pltpu.sync_copy(val_v, out_hbm.at[pl.ds(base, _LANES)])

    return gather_kernel


def _make_tc_count(batch, num_classes, blk):
    grid = (num_classes // blk,)

    def body(prob_ref, v_ref, out_ref, acc_ref):
        i = pl.program_id(0)

        @pl.when(i == 0)
        def _():
            acc_ref[...] = jnp.zeros_like(acc_ref)

        ge = (prob_ref[...] >= v_ref[...]).astype(jnp.float32)
        acc_ref[...] = acc_ref[...] + jnp.sum(ge, axis=1, keepdims=True)

        @pl.when(i == grid[0] - 1)
        def _():
            hits = (acc_ref[...] <= float(_K)).astype(jnp.float32)
            out_ref[0, 0] = jnp.sum(hits) / float(batch)

    return pl.pallas_call(
        body,
        grid=grid,
        in_specs=[
            pl.BlockSpec((batch, blk), lambda i: (0, i)),
            pl.BlockSpec((batch, 1), lambda i: (0, 0)),
        ],
        out_specs=pl.BlockSpec((1, 1), lambda i: (0, 0), memory_space=pltpu.SMEM),
        out_shape=jax.ShapeDtypeStruct((1, 1), jnp.float32),
        scratch_shapes=[pltpu.VMEM((batch, 1), jnp.float32)],
    )


@jax.jit
def kernel(prob, label):
    batch, num_classes = prob.shape
    v = _make_sc_gather(batch, num_classes)(prob, label)
    out = _make_tc_count(batch, num_classes, 16384)(prob, v.reshape(batch, 1))
    return out[0, 0]
